# baseline (speedup 1.0000x reference)
import functools

import jax
import jax.numpy as jnp
from jax import lax
from jax.experimental import pallas as pl
from jax.experimental.pallas import tpu as pltpu
from jax.experimental.pallas import tpu_sc as plsc

SEQ_LEN = 8192
BATCH = 2
D_MODEL = 1024

NC = 2
NS = 16
NW = NC * NS
ROWS_PER_W = SEQ_LEN // NW
TILE = 8
NT = ROWS_PER_W // TILE
NBUF = 4
PREFETCH = 2
NVEC = D_MODEL // 16


def _sc_body(x_hbm, t_hbm, out_hbm,
             a0, a1, a2, a3, b0, b1, b2, b3, t0, t1, t2, t3,
             sem_in_a, sem_in_b, sem_in_t, sem_out_a, sem_out_b):
    xa = (a0, a1, a2, a3)
    xb = (b0, b1, b2, b3)
    tv = (t0, t1, t2, t3)
    wid = lax.axis_index("s") * NC + lax.axis_index("c")
    base = wid * ROWS_PER_W

    def in_cps(g, b):
        rows = pl.ds(base + g * TILE, TILE)
        return (
            pltpu.make_async_copy(x_hbm.at[rows, 0], xa[b], sem_in_a.at[b]),
            pltpu.make_async_copy(x_hbm.at[rows, 1], xb[b], sem_in_b.at[b]),
            pltpu.make_async_copy(t_hbm.at[rows], tv[b], sem_in_t.at[b]),
        )

    def out_cps(g, b):
        rows = pl.ds(base + g * TILE, TILE)
        return (
            pltpu.make_async_copy(xa[b], out_hbm.at[rows, 0], sem_out_a.at[b]),
            pltpu.make_async_copy(xb[b], out_hbm.at[rows, 1], sem_out_b.at[b]),
        )

    def compute(b):
        va, vb, vt = xa[b], xb[b], tv[b]

        @plsc.parallel_loop(0, TILE, unroll=2)
        def _row(r):
            for v in range(NVEC):
                s = pl.ds(v * 16, 16)
                t = vt[r, s]
                plsc.addupdate(va.at[r, s], t)
                plsc.addupdate(vb.at[r, s], t)

    for p in range(PREFETCH):
        for c in in_cps(p, p):
            c.start()

    def step(gg, _):
        g0 = gg * NBUF
        for b in range(NBUF):
            g = g0 + b
            nxt = g + PREFETCH
            nb = (b + PREFETCH) % NBUF

            @pl.when(nxt < NT)
            def _():
                @pl.when(nxt >= NBUF)
                def _():
                    for c in out_cps(nxt - NBUF, nb):
                        c.wait()

                for c in in_cps(nxt, nb):
                    c.start()

            for c in in_cps(g, b):
                c.wait()
            compute(b)
            for c in out_cps(g, b):
                c.start()
        return 0

    lax.fori_loop(0, NT // NBUF, step, 0)

    for i in range(NBUF):
        for c in out_cps(NT - NBUF + i, i):
            c.wait()


def kernel(x, table):
    f = functools.partial(
        pl.kernel,
        mesh=plsc.VectorSubcoreMesh(core_axis_name="c", subcore_axis_name="s"),
        out_type=jax.ShapeDtypeStruct((SEQ_LEN, BATCH, D_MODEL), jnp.float32),
        scratch_types=(
            [pltpu.VMEM((TILE, D_MODEL), jnp.float32) for _ in range(3 * NBUF)]
            + [pltpu.SemaphoreType.DMA((NBUF,)) for _ in range(5)]
        ),
    )(_sc_body)
    return f(x, table)


# speedup vs baseline: 1.0406x; 1.0406x over previous
import functools

import jax
import jax.numpy as jnp
from jax import lax
from jax.experimental import pallas as pl
from jax.experimental.pallas import tpu as pltpu
from jax.experimental.pallas import tpu_sc as plsc

SEQ_LEN = 8192
BATCH = 2
D_MODEL = 1024

NC = 2
NS = 16
NW = NC * NS
ROWS_PER_W = SEQ_LEN // NW
TILE = 8
NT = ROWS_PER_W // TILE
NBUF = 4
PREFETCH = 2
NVEC = D_MODEL // 16


def _sc_body(x_hbm, t_hbm, out_hbm,
             a0, a1, a2, a3, b0, b1, b2, b3, t0, t1, t2, t3,
             sem_in_a, sem_in_b, sem_in_t, sem_out_a, sem_out_b):
    xa = (a0, a1, a2, a3)
    xb = (b0, b1, b2, b3)
    tv = (t0, t1, t2, t3)
    wid = lax.axis_index("s") * NC + lax.axis_index("c")
    base = wid * ROWS_PER_W

    def in_cps(g, b):
        rows = pl.ds(base + g * TILE, TILE)
        return (
            pltpu.make_async_copy(x_hbm.at[rows, 0], xa[b], sem_in_a.at[b]),
            pltpu.make_async_copy(x_hbm.at[rows, 1], xb[b], sem_in_b.at[b]),
            pltpu.make_async_copy(t_hbm.at[rows], tv[b], sem_in_t.at[b]),
        )

    def out_cps(g, b):
        rows = pl.ds(base + g * TILE, TILE)
        return (
            pltpu.make_async_copy(xa[b], out_hbm.at[rows, 0], sem_out_a.at[b]),
            pltpu.make_async_copy(xb[b], out_hbm.at[rows, 1], sem_out_b.at[b]),
        )

    def compute(b):
        va, vb, vt = xa[b], xb[b], tv[b]

        @plsc.parallel_loop(0, TILE, unroll=2)
        def _row(r):
            for v in range(NVEC):
                s = pl.ds(v * 16, 16)
                t = vt[r, s]
                va[r, s] = va[r, s] + t
                vb[r, s] = vb[r, s] + t

    for p in range(PREFETCH):
        for c in in_cps(p, p):
            c.start()

    def step(gg, _):
        g0 = gg * NBUF
        for b in range(NBUF):
            g = g0 + b
            nxt = g + PREFETCH
            nb = (b + PREFETCH) % NBUF

            @pl.when(nxt < NT)
            def _():
                @pl.when(nxt >= NBUF)
                def _():
                    for c in out_cps(nxt - NBUF, nb):
                        c.wait()

                for c in in_cps(nxt, nb):
                    c.start()

            for c in in_cps(g, b):
                c.wait()
            compute(b)
            for c in out_cps(g, b):
                c.start()
        return 0

    lax.fori_loop(0, NT // NBUF, step, 0)

    for i in range(NBUF):
        for c in out_cps(NT - NBUF + i, i):
            c.wait()


def kernel(x, table):
    f = functools.partial(
        pl.kernel,
        mesh=plsc.VectorSubcoreMesh(core_axis_name="c", subcore_axis_name="s"),
        out_type=jax.ShapeDtypeStruct((SEQ_LEN, BATCH, D_MODEL), jnp.float32),
        scratch_types=(
            [pltpu.VMEM((TILE, D_MODEL), jnp.float32) for _ in range(3 * NBUF)]
            + [pltpu.SemaphoreType.DMA((NBUF,)) for _ in range(5)]
        ),
    )(_sc_body)
    return f(x, table)


# speedup vs baseline: 1.2452x; 1.1966x over previous
import functools

import jax
import jax.numpy as jnp
from jax import lax
from jax.experimental import pallas as pl
from jax.experimental.pallas import tpu as pltpu
from jax.experimental.pallas import tpu_sc as plsc

SEQ_LEN = 8192
BATCH = 2
D_MODEL = 1024

NC = 2
NS = 16
NW = NC * NS
ROWS_PER_W = SEQ_LEN // NW
TILE = 8
NT = ROWS_PER_W // TILE
NBUF = 4
PREFETCH = 2
NVEC = D_MODEL // 16


def _sc_body(x_hbm, t_hbm, out_hbm,
             a0, a1, a2, a3, b0, b1, b2, b3, t0, t1, t2, t3,
             sem_in_a, sem_in_b, sem_in_t, sem_out_a, sem_out_b):
    xa = (a0, a1, a2, a3)
    xb = (b0, b1, b2, b3)
    tv = (t0, t1, t2, t3)
    wid = lax.axis_index("s") * NC + lax.axis_index("c")
    base = wid * ROWS_PER_W

    def in_cps(g, b):
        rows = pl.ds(base + g * TILE, TILE)
        return (
            pltpu.make_async_copy(x_hbm.at[rows, 0], xa[b], sem_in_a.at[b]),
            pltpu.make_async_copy(x_hbm.at[rows, 1], xb[b], sem_in_b.at[b]),
            pltpu.make_async_copy(t_hbm.at[rows], tv[b], sem_in_t.at[b]),
        )

    def out_cps(g, b):
        rows = pl.ds(base + g * TILE, TILE)
        return (
            pltpu.make_async_copy(xa[b], out_hbm.at[rows, 0], sem_out_a.at[b]),
            pltpu.make_async_copy(xb[b], out_hbm.at[rows, 1], sem_out_b.at[b]),
        )

    def compute(b):
        va, vb, vt = xa[b], xb[b], tv[b]

        @plsc.parallel_loop(0, TILE * NVEC, unroll=2)
        def _grp(j):
            r = j // NVEC
            v = j % NVEC
            s = pl.ds(v * 16, 16)
            t = vt[r, s]
            va[r, s] = va[r, s] + t
            vb[r, s] = vb[r, s] + t

    for p in range(PREFETCH):
        for c in in_cps(p, p):
            c.start()

    def step(gg, _):
        g0 = gg * NBUF
        for b in range(NBUF):
            g = g0 + b
            nxt = g + PREFETCH
            nb = (b + PREFETCH) % NBUF

            @pl.when(nxt < NT)
            def _():
                @pl.when(nxt >= NBUF)
                def _():
                    for c in out_cps(nxt - NBUF, nb):
                        c.wait()

                for c in in_cps(nxt, nb):
                    c.start()

            for c in in_cps(g, b):
                c.wait()
            compute(b)
            for c in out_cps(g, b):
                c.start()
        return 0

    lax.fori_loop(0, NT // NBUF, step, 0)

    for i in range(NBUF):
        for c in out_cps(NT - NBUF + i, i):
            c.wait()


def kernel(x, table):
    f = functools.partial(
        pl.kernel,
        mesh=plsc.VectorSubcoreMesh(core_axis_name="c", subcore_axis_name="s"),
        out_type=jax.ShapeDtypeStruct((SEQ_LEN, BATCH, D_MODEL), jnp.float32),
        scratch_types=(
            [pltpu.VMEM((TILE, D_MODEL), jnp.float32) for _ in range(3 * NBUF)]
            + [pltpu.SemaphoreType.DMA((NBUF,)) for _ in range(5)]
        ),
    )(_sc_body)
    return f(x, table)
